# baseline (speedup 1.0000x reference)
import functools

import jax
import jax.numpy as jnp
from jax import lax
from jax.experimental import pallas as pl
from jax.experimental.pallas import tpu as pltpu
from jax.experimental.pallas import tpu_sc as plsc


@functools.lru_cache(maxsize=None)
def _build_gather(n_tok: int, n_batch: int, dim: int):
    info = plsc.get_sparse_core_info()
    nlanes = info.num_lanes
    nw = info.num_cores * info.num_subcores
    assert n_batch % nw == 0
    stripe = n_batch // nw
    nh = 2
    chunk = stripe // nh
    assert chunk % nlanes == 0 and chunk % 128 == 0
    nsteps = n_tok * nh
    assert (nsteps - 3 - 4) % 3 == 0

    mesh = plsc.VectorSubcoreMesh(core_axis_name="c", subcore_axis_name="s")

    @functools.partial(
        pl.kernel,
        mesh=mesh,
        out_type=jax.ShapeDtypeStruct((n_tok, dim, n_batch), jnp.float32),
        scratch_types=[
            pltpu.VMEM((n_tok, stripe), jnp.int32),
            pltpu.VMEM((3, chunk, dim), jnp.float32),
            pltpu.VMEM((3, dim, chunk), jnp.float32),
            pltpu.SemaphoreType.DMA((3,)),
            pltpu.SemaphoreType.DMA((3,)),
        ],
        compiler_params=pltpu.CompilerParams(
            use_tc_tiling_on_sc=False, needs_layout_passes=False
        ),
    )
    def gather_kernel(idx_hbm, table_hbm, out_hbm, idx_v, rows_v, tbuf_v, gsem, ssem):
        wid = lax.axis_index("s") * info.num_cores + lax.axis_index("c")
        b0 = pl.multiple_of(wid * stripe, 128)

        pltpu.sync_copy(idx_hbm.at[:, pl.ds(b0, stripe)], idx_v)

        def fire_gather(s, buf):
            t = s // nh
            h = s % nh
            pltpu.async_copy(
                table_hbm.at[idx_v.at[t, pl.ds(h * chunk, chunk)]],
                rows_v.at[buf],
                gsem.at[buf],
            )

        def wait_gather(buf):
            pltpu.make_async_copy(
                table_hbm.at[pl.ds(0, chunk)], rows_v.at[buf], gsem.at[buf]
            ).wait()

        def transpose(buf):
            lanes = lax.iota(jnp.int32, nlanes)

            @plsc.parallel_loop(0, chunk // nlanes, unroll=4)
            def _(i):
                row_idx = lanes + i * nlanes
                for d in range(dim):
                    col_idx = jnp.full((nlanes,), d, jnp.int32)
                    vals = plsc.load_gather(rows_v.at[buf], [row_idx, col_idx])
                    tbuf_v[buf, d, pl.ds(i * nlanes, nlanes)] = vals

        def fire_store(s, buf):
            t = s // nh
            h = s % nh
            pltpu.async_copy(
                tbuf_v.at[buf],
                out_hbm.at[t, :, pl.ds(b0 + h * chunk, chunk)],
                ssem.at[buf],
            )

        def wait_store(buf):
            pltpu.make_async_copy(
                tbuf_v.at[buf], out_hbm.at[0, :, pl.ds(0, chunk)], ssem.at[buf]
            ).wait()

        def step(s, b, wait_st, fire_s):
            wait_gather(b)
            if wait_st:
                wait_store(b)
            transpose(b)
            fire_store(s, b)
            if fire_s is not None:
                fire_gather(fire_s, b)

        for b in range(3):
            fire_gather(b, b)
        for b in range(3):
            step(b, b, False, b + 3)

        @pl.loop(0, (nsteps - 7) // 3)
        def _(i):
            base = 3 + 3 * i
            for k in range(3):
                s = base + k
                step(s, k, True, s + 3)

        for s in range(nsteps - 4, nsteps):
            nxt = s + 3 if s + 3 < nsteps else None
            step(s, s % 3, True, nxt)
        for b in range(3):
            wait_store(b)

    return gather_kernel


def kernel(indices, table):
    b, t = indices.shape
    dim = table.shape[1]
    idx_t = jnp.swapaxes(indices, 0, 1).astype(jnp.int32)
    out = _build_gather(t, b, dim)(idx_t, table)
    return jnp.transpose(out, (2, 0, 1))


# speedup vs baseline: 1.0573x; 1.0573x over previous
import functools

import jax
import jax.numpy as jnp
from jax import lax
from jax.experimental import pallas as pl
from jax.experimental.pallas import tpu as pltpu
from jax.experimental.pallas import tpu_sc as plsc


@functools.lru_cache(maxsize=None)
def _build_gather(n_tok: int, n_batch: int, dim: int, n_grp: int):
    info = plsc.get_sparse_core_info()
    nlanes = info.num_lanes
    nw = info.num_cores * info.num_subcores
    assert n_batch % nw == 0
    stripe = n_batch // nw
    nh = 4
    chunk = stripe // nh
    assert chunk % nlanes == 0 and chunk % 128 == 0
    nsteps = n_tok * nh
    epi = 3 + (nsteps - 3) % 3
    nmain = (nsteps - 3 - epi) // 3
    rows_per_grp = 128 // dim
    assert rows_per_grp * dim == 128

    mesh = plsc.VectorSubcoreMesh(core_axis_name="c", subcore_axis_name="s")

    @functools.partial(
        pl.kernel,
        mesh=mesh,
        out_type=jax.ShapeDtypeStruct((n_tok, dim, n_batch), jnp.float32),
        scratch_types=[
            pltpu.VMEM((n_tok, stripe), jnp.int32),
            pltpu.VMEM((3, chunk), jnp.int32),
            pltpu.VMEM((3, chunk), jnp.int32),
            pltpu.VMEM((3, chunk, 128), jnp.float32),
            pltpu.VMEM((3, dim, chunk), jnp.float32),
            pltpu.SemaphoreType.DMA((3,)),
            pltpu.SemaphoreType.DMA((3,)),
        ],
        compiler_params=pltpu.CompilerParams(
            use_tc_tiling_on_sc=False, needs_layout_passes=False
        ),
    )
    def gather_kernel(
        idx_hbm, table_hbm, out_hbm, idx_v, g_v, r_v, rows_v, tbuf_v, gsem, ssem
    ):
        wid = lax.axis_index("s") * info.num_cores + lax.axis_index("c")
        b0 = pl.multiple_of(wid * stripe, 128)

        pltpu.sync_copy(idx_hbm.at[:, pl.ds(b0, stripe)], idx_v)

        def fire_gather(s, buf):
            t = s // nh
            h = s % nh

            @plsc.parallel_loop(0, chunk // nlanes, unroll=2)
            def _(i):
                v = idx_v[t, pl.ds(h * chunk + i * nlanes, nlanes)]
                g_v[buf, pl.ds(i * nlanes, nlanes)] = lax.shift_right_logical(
                    v, rows_per_grp // 2
                )
                r_v[buf, pl.ds(i * nlanes, nlanes)] = lax.shift_left(
                    lax.rem(v, rows_per_grp), 5
                )

            pltpu.async_copy(
                table_hbm.at[g_v.at[buf]], rows_v.at[buf], gsem.at[buf]
            )

        def wait_gather(buf):
            pltpu.make_async_copy(
                table_hbm.at[pl.ds(0, chunk)], rows_v.at[buf], gsem.at[buf]
            ).wait()

        def transpose(buf):
            lanes = lax.iota(jnp.int32, nlanes)

            @plsc.parallel_loop(0, chunk // nlanes, unroll=2)
            def _(i):
                row_idx = lanes + i * nlanes
                rvec = r_v[buf, pl.ds(i * nlanes, nlanes)]
                for d in range(dim):
                    col_idx = rvec + d
                    vals = plsc.load_gather(rows_v.at[buf], [row_idx, col_idx])
                    tbuf_v[buf, d, pl.ds(i * nlanes, nlanes)] = vals

        def fire_store(s, buf):
            t = s // nh
            h = s % nh
            pltpu.async_copy(
                tbuf_v.at[buf],
                out_hbm.at[t, :, pl.ds(b0 + h * chunk, chunk)],
                ssem.at[buf],
            )

        def wait_store(buf):
            pltpu.make_async_copy(
                tbuf_v.at[buf], out_hbm.at[0, :, pl.ds(0, chunk)], ssem.at[buf]
            ).wait()

        def step(s, b, wait_st, fire_s):
            wait_gather(b)
            if wait_st:
                wait_store(b)
            transpose(b)
            fire_store(s, b)
            if fire_s is not None:
                fire_gather(fire_s, b)

        for b in range(3):
            fire_gather(b, b)
        for b in range(3):
            step(b, b, False, b + 3)

        @pl.loop(0, nmain)
        def _(i):
            base = 3 + 3 * i
            for k in range(3):
                s = base + k
                step(s, k, True, s + 3)

        for s in range(nsteps - epi, nsteps):
            nxt = s + 3 if s + 3 < nsteps else None
            step(s, s % 3, True, nxt)
        for b in range(3):
            wait_store(b)

    return gather_kernel


def kernel(indices, table):
    b, t = indices.shape
    vocab, dim = table.shape
    idx_t = jnp.swapaxes(indices, 0, 1).astype(jnp.int32)
    n_grp = vocab * dim // 128
    table4 = table.reshape(n_grp, 128)
    out = _build_gather(t, b, dim, n_grp)(idx_t, table4)
    return jnp.transpose(out, (2, 0, 1))


# speedup vs baseline: 1.1983x; 1.1334x over previous
import functools

import jax
import jax.numpy as jnp
from jax import lax
from jax.experimental import pallas as pl
from jax.experimental.pallas import tpu as pltpu
from jax.experimental.pallas import tpu_sc as plsc


@functools.lru_cache(maxsize=None)
def _build_gather(n_tok: int, n_batch: int, dim: int, n_grp: int):
    info = plsc.get_sparse_core_info()
    nlanes = info.num_lanes
    nw = info.num_cores * info.num_subcores
    assert n_batch % nw == 0
    stripe = n_batch // nw
    nh = 4
    chunk = stripe // nh
    assert chunk % nlanes == 0 and chunk % 128 == 0
    nsteps = n_tok * nh
    epi = 3 + (nsteps - 3) % 3
    nmain = (nsteps - 3 - epi) // 3
    rows_per_grp = 128 // dim
    assert rows_per_grp * dim == 128

    mesh = plsc.VectorSubcoreMesh(core_axis_name="c", subcore_axis_name="s")

    @functools.partial(
        pl.kernel,
        mesh=mesh,
        out_type=jax.ShapeDtypeStruct((n_tok, dim, n_batch), jnp.float32),
        scratch_types=[
            pltpu.VMEM((n_tok, stripe), jnp.int32),
            pltpu.VMEM((3, chunk), jnp.int32),
            pltpu.VMEM((3, chunk), jnp.int32),
            pltpu.VMEM((3, chunk, 128), jnp.float32),
            pltpu.VMEM((3, dim, chunk), jnp.float32),
            pltpu.SemaphoreType.DMA((3,)),
            pltpu.SemaphoreType.DMA((3,)),
        ],
        compiler_params=pltpu.CompilerParams(
            use_tc_tiling_on_sc=True, needs_layout_passes=False
        ),
    )
    def gather_kernel(
        idx_hbm, table_hbm, out_hbm, idx_v, g_v, r_v, rows_v, tbuf_v, gsem, ssem
    ):
        wid = lax.axis_index("s") * info.num_cores + lax.axis_index("c")
        b0 = pl.multiple_of(wid * stripe, 128)

        pltpu.sync_copy(idx_hbm.at[:, pl.ds(b0, stripe)], idx_v)

        def fire_gather(s, buf):
            t = s // nh
            h = s % nh

            @plsc.parallel_loop(0, chunk // nlanes, unroll=2)
            def _(i):
                v = idx_v[t, pl.ds(h * chunk + i * nlanes, nlanes)]
                g_v[buf, pl.ds(i * nlanes, nlanes)] = lax.shift_right_logical(
                    v, rows_per_grp // 2
                )
                r_v[buf, pl.ds(i * nlanes, nlanes)] = lax.shift_left(
                    lax.rem(v, rows_per_grp), 5
                )

            pltpu.async_copy(
                table_hbm.at[g_v.at[buf]], rows_v.at[buf], gsem.at[buf]
            )

        def wait_gather(buf):
            pltpu.make_async_copy(
                table_hbm.at[pl.ds(0, chunk)], rows_v.at[buf], gsem.at[buf]
            ).wait()

        def transpose(buf):
            lanes = lax.iota(jnp.int32, nlanes)

            @plsc.parallel_loop(0, chunk // nlanes, unroll=2)
            def _(i):
                row_idx = lanes + i * nlanes
                rvec = r_v[buf, pl.ds(i * nlanes, nlanes)]
                for d in range(dim):
                    col_idx = rvec + d
                    vals = plsc.load_gather(rows_v.at[buf], [row_idx, col_idx])
                    tbuf_v[buf, d, pl.ds(i * nlanes, nlanes)] = vals

        def fire_store(s, buf):
            t = s // nh
            h = s % nh
            pltpu.async_copy(
                tbuf_v.at[buf],
                out_hbm.at[t, :, pl.ds(b0 + h * chunk, chunk)],
                ssem.at[buf],
            )

        def wait_store(buf):
            pltpu.make_async_copy(
                tbuf_v.at[buf], out_hbm.at[0, :, pl.ds(0, chunk)], ssem.at[buf]
            ).wait()

        def step(s, b, wait_st, fire_s):
            wait_gather(b)
            if wait_st:
                wait_store(b)
            transpose(b)
            fire_store(s, b)
            if fire_s is not None:
                fire_gather(fire_s, b)

        for b in range(3):
            fire_gather(b, b)
        for b in range(3):
            step(b, b, False, b + 3)

        @pl.loop(0, nmain)
        def _(i):
            base = 3 + 3 * i
            for k in range(3):
                s = base + k
                step(s, k, True, s + 3)

        for s in range(nsteps - epi, nsteps):
            nxt = s + 3 if s + 3 < nsteps else None
            step(s, s % 3, True, nxt)
        for b in range(3):
            wait_store(b)

    return gather_kernel


def kernel(indices, table):
    b, t = indices.shape
    vocab, dim = table.shape
    idx_t = jnp.swapaxes(indices, 0, 1).astype(jnp.int32)
    n_grp = vocab * dim // 128
    table4 = table.reshape(n_grp, 128)
    out = _build_gather(t, b, dim, n_grp)(idx_t, table4)
    return jnp.transpose(out, (2, 0, 1))
